# Initial kernel scaffold; baseline (speedup 1.0000x reference)
import jax
import jax.numpy as jnp
from jax.experimental import pallas as pl


def kernel(s, v, p, edge_index_local, d_local, a_local, r_norm_local, e_local, edge_index_global, d_global, a_global, r_norm_global, e_global, W1, b1, W2, b2, gamma, beta, Wu1, bu1, Wu2, bu2):
    raise NotImplementedError("write your pallas kernel here")


# speedup vs baseline: 19.0206x; 19.0206x over previous
import functools

import jax
import jax.numpy as jnp
from jax import lax
from jax.experimental import pallas as pl
from jax.experimental.pallas import tpu as pltpu
from jax.experimental.pallas import tpu_sc as plsc

NN = 50000
EE = 800000
SDIM = 64
VFD = 48
EDIM = 16
HID = 64
PP = 8
PMW = 4

NCORES = 2
NSUB = 16
NWORK = NCORES * NSUB

BN = 1000
BE = 2000
GC = 1000
SCC = 400

EPW = EE // NWORK
GITERS = EPW // GC
EPT = EE // NSUB
SITERS = EPT // SCC
NPT = NN // NSUB

_f32 = jnp.float32


def _norm_body(s_ref, v_ref, g_ref, b_ref, so_ref, vo_ref):
    s = s_ref[...]
    mu = jnp.mean(s, axis=-1, keepdims=True)
    var = jnp.mean((s - mu) ** 2, axis=-1, keepdims=True)
    so_ref[...] = g_ref[...] * (s - mu) / jnp.sqrt(var + 1e-6) + b_ref[...]
    v = v_ref[...]
    vn = jnp.sqrt(jnp.sum(v * v, axis=-1, keepdims=True) / 16.0 + 1e-6)
    vo_ref[...] = v / vn


def _norm_nodes(s, v, gamma_i, beta_i):
    grid = (NN // BN,)
    return pl.pallas_call(
        _norm_body,
        grid=grid,
        in_specs=[
            pl.BlockSpec((BN, SDIM), lambda i: (i, 0)),
            pl.BlockSpec((BN, VFD), lambda i: (i, 0)),
            pl.BlockSpec((1, SDIM), lambda i: (0, 0)),
            pl.BlockSpec((1, SDIM), lambda i: (0, 0)),
        ],
        out_specs=[
            pl.BlockSpec((BN, SDIM), lambda i: (i, 0)),
            pl.BlockSpec((BN, VFD), lambda i: (i, 0)),
        ],
        out_shape=[
            jax.ShapeDtypeStruct((NN, SDIM), _f32),
            jax.ShapeDtypeStruct((NN, VFD), _f32),
        ],
    )(s, v, gamma_i.reshape(1, SDIM), beta_i.reshape(1, SDIM))


def _gather_body(with_p, sn_hbm, vn_hbm, p_hbm, src_hbm, dst_hbm,
                 ssrc_o, sdst_o, vsrc_o, psrc_o, pdst_o,
                 isrc_v, idst_v, b64, b48, b8, sem):
    wid = lax.axis_index("s") * NCORES + lax.axis_index("c")
    base0 = wid * EPW

    def chunk(ci, _):
        base = base0 + ci * GC
        pltpu.sync_copy(src_hbm.at[pl.ds(base, GC)], isrc_v)
        pltpu.sync_copy(dst_hbm.at[pl.ds(base, GC)], idst_v)
        pltpu.async_copy(sn_hbm.at[isrc_v], b64, sem).wait()
        pltpu.sync_copy(b64, ssrc_o.at[pl.ds(base, GC)])
        pltpu.async_copy(sn_hbm.at[idst_v], b64, sem).wait()
        pltpu.sync_copy(b64, sdst_o.at[pl.ds(base, GC)])
        pltpu.async_copy(vn_hbm.at[isrc_v], b48, sem).wait()
        pltpu.sync_copy(b48, vsrc_o.at[pl.ds(base, GC)])
        if with_p:
            pltpu.async_copy(p_hbm.at[isrc_v], b8, sem).wait()
            pltpu.sync_copy(b8, psrc_o.at[pl.ds(base, GC)])
            pltpu.async_copy(p_hbm.at[idst_v], b8, sem).wait()
            pltpu.sync_copy(b8, pdst_o.at[pl.ds(base, GC)])

    pl.loop(0, GITERS)(lambda i: chunk(i, None))


def _gather_edges(sn, vn, p8, src, dst, with_p):
    mesh = plsc.VectorSubcoreMesh(core_axis_name="c", subcore_axis_name="s")
    out_type = [
        jax.ShapeDtypeStruct((EE, SDIM), _f32),
        jax.ShapeDtypeStruct((EE, SDIM), _f32),
        jax.ShapeDtypeStruct((EE, VFD), _f32),
        jax.ShapeDtypeStruct((EE, PP), _f32),
        jax.ShapeDtypeStruct((EE, PP), _f32),
    ]
    k = pl.kernel(
        functools.partial(_gather_body, with_p),
        out_type=out_type,
        mesh=mesh,
        compiler_params=pltpu.CompilerParams(use_tc_tiling_on_sc=False),
        scratch_types=[
            pltpu.VMEM((GC,), jnp.int32),
            pltpu.VMEM((GC,), jnp.int32),
            pltpu.VMEM((GC, SDIM), _f32),
            pltpu.VMEM((GC, VFD), _f32),
            pltpu.VMEM((GC, PP), _f32),
            pltpu.SemaphoreType.DMA,
        ],
    )
    return k(sn, vn, p8, src, dst)


def _edge_body(first, ssrc_ref, sdst_ref, vsrc_ref, e_ref,
               da1_ref, da2_ref,
               w1a_ref, w1b_ref, w1c_ref, w1da_ref, b1_ref,
               w2_ref, b2_ref,
               sm_o, vm_o, en_o):
    if first:
        x1 = da1_ref[...]
        d = x1[:, 0:1]
        a = x1[:, 1:2]
        rn3 = x1[:, 2:5]
    else:
        psrc = da1_ref[...]
        pdst = da2_ref[...]
        r = pdst - psrc
        a = jnp.sum(pdst * psrc, axis=-1, keepdims=True)
        d2 = jnp.sum(r * r, axis=-1, keepdims=True)
        d = jnp.sqrt(jnp.maximum(d2, 1e-6))
        rn3 = (r / (1.0 + d))[:, 0:3]

    h = jnp.dot(ssrc_ref[...], w1a_ref[...], preferred_element_type=_f32)
    h = h + jnp.dot(sdst_ref[...], w1b_ref[...], preferred_element_type=_f32)
    h = h + jnp.dot(e_ref[...], w1c_ref[...], preferred_element_type=_f32)
    da = jnp.concatenate([d, a], axis=-1)
    h = h + jnp.dot(da, w1da_ref[...], preferred_element_type=_f32)
    h = h + b1_ref[...]
    h = h * jax.nn.sigmoid(h)
    out = jnp.dot(h, w2_ref[...], preferred_element_type=_f32) + b2_ref[...]

    s_msg = out[:, 0:SDIM]
    w_vv = out[:, SDIM:SDIM + 16]
    w_vs = out[:, SDIM + 16:SDIM + 32]
    e_new = out[:, SDIM + 32:SDIM + 48]
    w_p = out[:, SDIM + 48:SDIM + 49]

    sm_o[...] = jnp.stack([s_msg[:, 0:32], s_msg[:, 32:64]], axis=0)
    vsrc = vsrc_ref[...]
    vm = jnp.concatenate(
        [vsrc[:, 16 * k:16 * (k + 1)] * w_vv + rn3[:, k:k + 1] * w_vs
         for k in range(3)], axis=-1)
    tp = rn3 * jnp.tanh(w_p)
    ones = jnp.ones_like(w_p)
    z4 = jnp.zeros((tp.shape[0], 4), _f32)
    z8 = jnp.zeros((tp.shape[0], 8), _f32)
    vm_o[...] = jnp.stack([
        jnp.concatenate([vm[:, 0:24], tp, ones, z4], axis=-1),
        jnp.concatenate([vm[:, 24:48], z8], axis=-1)], axis=0)
    en_o[...] = e_new


def _edge_mlp(first, ssrc, sdst, vsrc, e, x1, x2,
              w1a, w1b, w1c, w1da, b1, w2, b2):
    grid = (EE // BE,)
    xw = x1.shape[1]
    kb = functools.partial(_edge_body, first)
    return pl.pallas_call(
        kb,
        grid=grid,
        in_specs=[
            pl.BlockSpec((BE, SDIM), lambda i: (i, 0)),
            pl.BlockSpec((BE, SDIM), lambda i: (i, 0)),
            pl.BlockSpec((BE, VFD), lambda i: (i, 0)),
            pl.BlockSpec((BE, EDIM), lambda i: (i, 0)),
            pl.BlockSpec((BE, xw), lambda i: (i, 0)),
            pl.BlockSpec((BE, xw), lambda i: (i, 0)),
            pl.BlockSpec((SDIM, HID), lambda i: (0, 0)),
            pl.BlockSpec((SDIM, HID), lambda i: (0, 0)),
            pl.BlockSpec((EDIM, HID), lambda i: (0, 0)),
            pl.BlockSpec((2, HID), lambda i: (0, 0)),
            pl.BlockSpec((1, HID), lambda i: (0, 0)),
            pl.BlockSpec((HID, 128), lambda i: (0, 0)),
            pl.BlockSpec((1, 128), lambda i: (0, 0)),
        ],
        out_specs=[
            pl.BlockSpec((2, BE, 32), lambda i: (0, i, 0)),
            pl.BlockSpec((2, BE, 32), lambda i: (0, i, 0)),
            pl.BlockSpec((BE, EDIM), lambda i: (i, 0)),
        ],
        out_shape=[
            jax.ShapeDtypeStruct((2, EE, 32), _f32),
            jax.ShapeDtypeStruct((2, EE, 32), _f32),
            jax.ShapeDtypeStruct((EE, EDIM), _f32),
        ],
    )(ssrc, sdst, vsrc, e, x1, x2, w1a, w1b, w1c, w1da, b1, w2, b2)


def _scat_s_body(sn_hbm, sm_hbm, dst_hbm, out_hbm,
                 idx_v, msg_v, acc, sem):
    cid = lax.axis_index("c")
    sid = lax.axis_index("s")
    r0 = sid * NPT
    pltpu.sync_copy(sn_hbm.at[pl.ds(r0, NPT), pl.ds(cid * 32, 32)],
                    acc.at[pl.ds(r0, NPT)])
    plsc.subcore_barrier()

    def chunk(ci):
        base = sid * EPT + ci * SCC
        pltpu.sync_copy(dst_hbm.at[pl.ds(base, SCC)], idx_v)
        pltpu.sync_copy(sm_hbm.at[cid, pl.ds(base, SCC)], msg_v)
        pltpu.sync_copy(msg_v, acc.at[idx_v], add=True)

    pl.loop(0, SITERS)(chunk)
    plsc.subcore_barrier()
    pltpu.sync_copy(acc.at[pl.ds(r0, NPT)], out_hbm.at[cid, pl.ds(r0, NPT)])


def _scatter_s(sn, sm, dst):
    mesh = plsc.VectorSubcoreMesh(core_axis_name="c", subcore_axis_name="s")
    k = pl.kernel(
        _scat_s_body,
        out_type=jax.ShapeDtypeStruct((2, NN, 32), _f32),
        mesh=mesh,
        compiler_params=pltpu.CompilerParams(use_tc_tiling_on_sc=False),
        scratch_types=[
            pltpu.VMEM((SCC,), jnp.int32),
            pltpu.VMEM((SCC, 32), _f32),
            pltpu.VMEM_SHARED((NN, 32), _f32),
            pltpu.SemaphoreType.DMA,
        ],
    )
    return k(sn, sm, dst)


def _scat_vp_body(vm_hbm, dst_hbm, zv_hbm, outv_hbm,
                  idx_v, msgv_v, accv, sem):
    cid = lax.axis_index("c")
    sid = lax.axis_index("s")
    r0 = sid * NPT
    pltpu.sync_copy(zv_hbm.at[pl.ds(r0, NPT)], accv.at[pl.ds(r0, NPT)])
    plsc.subcore_barrier()

    def chunk(ci):
        base = sid * EPT + ci * SCC
        pltpu.sync_copy(dst_hbm.at[pl.ds(base, SCC)], idx_v)
        pltpu.sync_copy(vm_hbm.at[cid, pl.ds(base, SCC)], msgv_v)
        pltpu.sync_copy(msgv_v, accv.at[idx_v], add=True)

    pl.loop(0, SITERS)(chunk)
    plsc.subcore_barrier()
    pltpu.sync_copy(accv.at[pl.ds(r0, NPT)], outv_hbm.at[cid, pl.ds(r0, NPT)])


def _scatter_vp(vm, dst, zv):
    mesh = plsc.VectorSubcoreMesh(core_axis_name="c", subcore_axis_name="s")
    k = pl.kernel(
        _scat_vp_body,
        out_type=jax.ShapeDtypeStruct((2, NN, 32), _f32),
        mesh=mesh,
        compiler_params=pltpu.CompilerParams(use_tc_tiling_on_sc=False),
        scratch_types=[
            pltpu.VMEM((SCC,), jnp.int32),
            pltpu.VMEM((SCC, 32), _f32),
            pltpu.VMEM_SHARED((NN, 32), _f32),
            pltpu.SemaphoreType.DMA,
        ],
    )
    return k(vm, dst, zv)


def _combine_body(do_mlp, sa_ref, vn_ref, vs_ref,
                  p_ref, wu1_ref, bu1_ref, wu2_ref, bu2_ref,
                  so_ref, vo_ref, po_ref):
    vs0 = vs_ref[0]
    vs1 = vs_ref[1]
    cnt = jnp.maximum(vs0[:, 27:28], 1.0)
    vsum = jnp.concatenate([vs0[:, 0:24], vs1[:, 0:24]], axis=-1)
    vo_ref[...] = vn_ref[...] + vsum / cnt
    pupd = jnp.concatenate(
        [vs0[:, 24:27] / cnt, jnp.zeros((vs0.shape[0], PP - 3), _f32)],
        axis=-1)
    po_ref[...] = p_ref[...] + pupd
    s = jnp.concatenate([sa_ref[0], sa_ref[1]], axis=-1)
    if do_mlp:
        h = jnp.dot(s, wu1_ref[...], preferred_element_type=_f32) + bu1_ref[...]
        h = h * jax.nn.sigmoid(h)
        s = s + jnp.dot(h, wu2_ref[...], preferred_element_type=_f32) + bu2_ref[...]
    so_ref[...] = s


def _combine(do_mlp, sagg, vn, vsum, p8, wu1, bu1, wu2, bu2):
    grid = (NN // BN,)
    kb = functools.partial(_combine_body, do_mlp)
    return pl.pallas_call(
        kb,
        grid=grid,
        in_specs=[
            pl.BlockSpec((2, BN, 32), lambda i: (0, i, 0)),
            pl.BlockSpec((BN, VFD), lambda i: (i, 0)),
            pl.BlockSpec((2, BN, 32), lambda i: (0, i, 0)),
            pl.BlockSpec((BN, PP), lambda i: (i, 0)),
            pl.BlockSpec((SDIM, HID), lambda i: (0, 0)),
            pl.BlockSpec((1, HID), lambda i: (0, 0)),
            pl.BlockSpec((HID, SDIM), lambda i: (0, 0)),
            pl.BlockSpec((1, SDIM), lambda i: (0, 0)),
        ],
        out_specs=[
            pl.BlockSpec((BN, SDIM), lambda i: (i, 0)),
            pl.BlockSpec((BN, VFD), lambda i: (i, 0)),
            pl.BlockSpec((BN, PP), lambda i: (i, 0)),
        ],
        out_shape=[
            jax.ShapeDtypeStruct((NN, SDIM), _f32),
            jax.ShapeDtypeStruct((NN, VFD), _f32),
            jax.ShapeDtypeStruct((NN, PP), _f32),
        ],
    )(sagg, vn, vsum, p8, wu1, bu1, wu2, bu2)


def kernel(s, v, p, edge_index_local, d_local, a_local, r_norm_local, e_local,
           edge_index_global, d_global, a_global, r_norm_global, e_global,
           W1, b1, W2, b2, gamma, beta, Wu1, bu1, Wu2, bu2):
    nl = W1.shape[0]
    src = edge_index_global[0].astype(jnp.int32)
    dst = edge_index_global[1].astype(jnp.int32)

    v = v.reshape(NN, VFD)
    p8 = jnp.pad(p, ((0, 0), (0, PP - 3)))

    w1a = W1[:, 0:SDIM, :]
    w1b = W1[:, SDIM:2 * SDIM, :]
    w1c = W1[:, 2 * SDIM:2 * SDIM + EDIM, :]
    w1da = W1[:, 2 * SDIM + EDIM:, :]
    sd, vd = SDIM, 16
    w2r = jnp.concatenate([
        W2[:, :, 0:sd],
        W2[:, :, sd:sd + vd],
        W2[:, :, sd + vd:sd + 2 * vd],
        W2[:, :, sd + 2 * vd + 1:],
        W2[:, :, sd + 2 * vd:sd + 2 * vd + 1],
        jnp.zeros((nl, HID, 128 - 113), _f32),
    ], axis=-1)
    b2r = jnp.concatenate([
        b2[:, 0:sd], b2[:, sd:sd + vd], b2[:, sd + vd:sd + 2 * vd],
        b2[:, sd + 2 * vd + 1:], b2[:, sd + 2 * vd:sd + 2 * vd + 1],
        jnp.zeros((nl, 128 - 113), _f32),
    ], axis=-1)

    zv = jnp.zeros((NN, 32), _f32)

    e = e_global
    da1 = jnp.concatenate(
        [d_global[:, None], a_global[:, None], r_norm_global,
         jnp.zeros((EE, 3), _f32)], axis=-1)

    for i in range(nl):
        sn, vn = _norm_nodes(s, v, gamma[i], beta[i])
        ssrc, sdst, vsrc, psrc, pdst = _gather_edges(
            sn, vn, p8, src, dst, with_p=(i > 0))
        if i == 0:
            x1, x2 = da1, da1
        else:
            x1, x2 = psrc, pdst
        sm, vm, e = _edge_mlp(
            i == 0, ssrc, sdst, vsrc, e, x1, x2,
            w1a[i], w1b[i], w1c[i], w1da[i], b1[i].reshape(1, HID),
            w2r[i], b2r[i].reshape(1, 128))
        sagg = _scatter_s(sn, sm, dst)
        vsum = _scatter_vp(vm, dst, zv)
        s, v, p8 = _combine(
            i < nl - 1, sagg, vn, vsum, p8,
            Wu1[i], bu1[i].reshape(1, HID), Wu2[i], bu2[i].reshape(1, SDIM))

    return (s, v.reshape(NN, 3, 16), e, p8[:, 0:3])


# speedup vs baseline: 24.1382x; 1.2691x over previous
import functools

import jax
import jax.numpy as jnp
from jax import lax
from jax.experimental import pallas as pl
from jax.experimental.pallas import tpu as pltpu
from jax.experimental.pallas import tpu_sc as plsc

NN = 50000
EE = 800000
SDIM = 64
VFD = 48
EDIM = 16
HID = 64
PP = 8
PK = 128

NCORES = 2
NSUB = 16
NWORK = NCORES * NSUB

BN = 1000
BE = 2000
GC = 1000
SCC = 400

EPW = EE // NWORK
GITERS = EPW // GC
EPT = EE // NSUB
SITERS = EPT // SCC
NPT = NN // NSUB

_f32 = jnp.float32


def _norm_pack(s, v, p8, gamma_i, beta_i):
    sn = gamma_i * (s - jnp.mean(s, axis=-1, keepdims=True)) / jnp.sqrt(
        jnp.mean((s - jnp.mean(s, axis=-1, keepdims=True)) ** 2,
                 axis=-1, keepdims=True) + 1e-6) + beta_i
    vn = v / jnp.sqrt(jnp.sum(v * v, axis=-1, keepdims=True) / 16.0 + 1e-6)
    return sn, vn


def _pack0_body(s_ref, v_ref, p_ref, g_ref, b_ref, pk_ref):
    sn, vn = _norm_pack(s_ref[...], v_ref[...], None, g_ref[...], b_ref[...])
    z = jnp.zeros((sn.shape[0], PK - SDIM - VFD - PP), _f32)
    pk_ref[...] = jnp.concatenate([sn, vn, p_ref[...], z], axis=-1)


def _pack0(s, v, p8, gamma_i, beta_i):
    return pl.pallas_call(
        _pack0_body,
        grid=(NN // BN,),
        in_specs=[
            pl.BlockSpec((BN, SDIM), lambda i: (i, 0)),
            pl.BlockSpec((BN, VFD), lambda i: (i, 0)),
            pl.BlockSpec((BN, PP), lambda i: (i, 0)),
            pl.BlockSpec((1, SDIM), lambda i: (0, 0)),
            pl.BlockSpec((1, SDIM), lambda i: (0, 0)),
        ],
        out_specs=pl.BlockSpec((BN, PK), lambda i: (i, 0)),
        out_shape=jax.ShapeDtypeStruct((NN, PK), _f32),
    )(s, v, p8, gamma_i.reshape(1, SDIM), beta_i.reshape(1, SDIM))


def _gather_body(pk_hbm, src_hbm, dst_hbm, srow_o, drow_o,
                 isrc_v, idst_v, brow, sem):
    wid = lax.axis_index("s") * NCORES + lax.axis_index("c")
    base0 = wid * EPW

    def chunk(ci):
        base = base0 + ci * GC
        pltpu.sync_copy(src_hbm.at[pl.ds(base, GC)], isrc_v)
        pltpu.sync_copy(dst_hbm.at[pl.ds(base, GC)], idst_v)
        pltpu.async_copy(pk_hbm.at[isrc_v], brow, sem).wait()
        pltpu.sync_copy(brow, srow_o.at[pl.ds(base, GC)])
        pltpu.async_copy(pk_hbm.at[idst_v], brow, sem).wait()
        pltpu.sync_copy(brow, drow_o.at[pl.ds(base, GC)])

    pl.loop(0, GITERS)(chunk)


def _gather_edges(pk, src, dst):
    mesh = plsc.VectorSubcoreMesh(core_axis_name="c", subcore_axis_name="s")
    k = pl.kernel(
        _gather_body,
        out_type=[
            jax.ShapeDtypeStruct((EE, PK), _f32),
            jax.ShapeDtypeStruct((EE, PK), _f32),
        ],
        mesh=mesh,
        scratch_types=[
            pltpu.VMEM((GC,), jnp.int32),
            pltpu.VMEM((GC,), jnp.int32),
            pltpu.VMEM((GC, PK), _f32),
            pltpu.SemaphoreType.DMA,
        ],
    )
    return k(pk, src, dst)


def _edge_body(first, srow_ref, drow_ref, e_ref, da_ref,
               w1s_ref, w1d_ref, w1c_ref, w1da_ref, b1_ref,
               w2_ref, b2_ref,
               sm_o, vm_o, en_o):
    srow = srow_ref[...]
    drow = drow_ref[...]
    if first:
        x1 = da_ref[...]
        d = x1[:, 0:1]
        a = x1[:, 1:2]
        rn3 = x1[:, 2:5]
    else:
        psrc = srow[:, 112:120]
        pdst = drow[:, 112:120]
        r = pdst - psrc
        a = jnp.sum(pdst * psrc, axis=-1, keepdims=True)
        d2 = jnp.sum(r * r, axis=-1, keepdims=True)
        d = jnp.sqrt(jnp.maximum(d2, 1e-6))
        rn3 = (r / (1.0 + d))[:, 0:3]

    h = jnp.dot(srow, w1s_ref[...], preferred_element_type=_f32)
    h = h + jnp.dot(drow, w1d_ref[...], preferred_element_type=_f32)
    h = h + jnp.dot(e_ref[...], w1c_ref[...], preferred_element_type=_f32)
    da = jnp.concatenate([d, a], axis=-1)
    h = h + jnp.dot(da, w1da_ref[...], preferred_element_type=_f32)
    h = h + b1_ref[...]
    h = h * jax.nn.sigmoid(h)
    out = jnp.dot(h, w2_ref[...], preferred_element_type=_f32) + b2_ref[...]

    s_msg = out[:, 0:SDIM]
    w_vv = out[:, SDIM:SDIM + 16]
    w_vs = out[:, SDIM + 16:SDIM + 32]
    e_new = out[:, SDIM + 32:SDIM + 48]
    w_p = out[:, SDIM + 48:SDIM + 49]

    sm_o[...] = jnp.stack([s_msg[:, 0:32], s_msg[:, 32:64]], axis=0)
    vsrc = srow[:, SDIM:SDIM + VFD]
    vm = jnp.concatenate(
        [vsrc[:, 16 * k:16 * (k + 1)] * w_vv + rn3[:, k:k + 1] * w_vs
         for k in range(3)], axis=-1)
    tp = rn3 * jnp.tanh(w_p)
    ones = jnp.ones_like(w_p)
    z4 = jnp.zeros((tp.shape[0], 4), _f32)
    z8 = jnp.zeros((tp.shape[0], 8), _f32)
    vm_o[...] = jnp.stack([
        jnp.concatenate([vm[:, 0:24], tp, ones, z4], axis=-1),
        jnp.concatenate([vm[:, 24:48], z8], axis=-1)], axis=0)
    en_o[...] = e_new


def _edge_mlp(first, srow, drow, e, da,
              w1s, w1d, w1c, w1da, b1, w2, b2):
    kb = functools.partial(_edge_body, first)
    return pl.pallas_call(
        kb,
        grid=(EE // BE,),
        in_specs=[
            pl.BlockSpec((BE, PK), lambda i: (i, 0)),
            pl.BlockSpec((BE, PK), lambda i: (i, 0)),
            pl.BlockSpec((BE, EDIM), lambda i: (i, 0)),
            pl.BlockSpec((BE, PP), lambda i: (i, 0)),
            pl.BlockSpec((PK, HID), lambda i: (0, 0)),
            pl.BlockSpec((PK, HID), lambda i: (0, 0)),
            pl.BlockSpec((EDIM, HID), lambda i: (0, 0)),
            pl.BlockSpec((2, HID), lambda i: (0, 0)),
            pl.BlockSpec((1, HID), lambda i: (0, 0)),
            pl.BlockSpec((HID, 128), lambda i: (0, 0)),
            pl.BlockSpec((1, 128), lambda i: (0, 0)),
        ],
        out_specs=[
            pl.BlockSpec((2, BE, 32), lambda i: (0, i, 0)),
            pl.BlockSpec((2, BE, 32), lambda i: (0, i, 0)),
            pl.BlockSpec((BE, EDIM), lambda i: (i, 0)),
        ],
        out_shape=[
            jax.ShapeDtypeStruct((2, EE, 32), _f32),
            jax.ShapeDtypeStruct((2, EE, 32), _f32),
            jax.ShapeDtypeStruct((EE, EDIM), _f32),
        ],
    )(srow, drow, e, da, w1s, w1d, w1c, w1da, b1, w2, b2)


def _scat_body(msg_hbm, dst_hbm, z_hbm, out_hbm,
               idx_v, msg_v, acc, sem):
    cid = lax.axis_index("c")
    sid = lax.axis_index("s")
    r0 = sid * NPT
    pltpu.sync_copy(z_hbm.at[pl.ds(r0, NPT)], acc.at[pl.ds(r0, NPT)])
    plsc.subcore_barrier()

    def chunk(ci):
        base = sid * EPT + ci * SCC
        pltpu.sync_copy(dst_hbm.at[pl.ds(base, SCC)], idx_v)
        pltpu.sync_copy(msg_hbm.at[cid, pl.ds(base, SCC)], msg_v)
        pltpu.sync_copy(msg_v, acc.at[idx_v], add=True)

    pl.loop(0, SITERS)(chunk)
    plsc.subcore_barrier()
    pltpu.sync_copy(acc.at[pl.ds(r0, NPT)], out_hbm.at[cid, pl.ds(r0, NPT)])


def _scatter32(msg, dst, z):
    mesh = plsc.VectorSubcoreMesh(core_axis_name="c", subcore_axis_name="s")
    k = pl.kernel(
        _scat_body,
        out_type=jax.ShapeDtypeStruct((2, NN, 32), _f32),
        mesh=mesh,
        compiler_params=pltpu.CompilerParams(use_tc_tiling_on_sc=False),
        scratch_types=[
            pltpu.VMEM((SCC,), jnp.int32),
            pltpu.VMEM((SCC, 32), _f32),
            pltpu.VMEM_SHARED((NN, 32), _f32),
            pltpu.SemaphoreType.DMA,
        ],
    )
    return k(msg, dst, z)


def _combine_body(last, pk_ref, ss_ref, vs_ref,
                  wu1_ref, bu1_ref, wu2_ref, bu2_ref, g_ref, b_ref,
                  so_ref, vo_ref, x_ref):
    pk = pk_ref[...]
    vs0 = vs_ref[0]
    vs1 = vs_ref[1]
    cnt = jnp.maximum(vs0[:, 27:28], 1.0)
    vsum = jnp.concatenate([vs0[:, 0:24], vs1[:, 0:24]], axis=-1)
    v_new = pk[:, SDIM:SDIM + VFD] + vsum / cnt
    p_new = jnp.concatenate(
        [pk[:, 112:115] + vs0[:, 24:27] / cnt,
         jnp.zeros((pk.shape[0], PP - 3), _f32)], axis=-1)
    s = pk[:, 0:SDIM] + jnp.concatenate([ss_ref[0], ss_ref[1]], axis=-1)
    if not last:
        h = jnp.dot(s, wu1_ref[...], preferred_element_type=_f32) + bu1_ref[...]
        h = h * jax.nn.sigmoid(h)
        s = s + jnp.dot(h, wu2_ref[...], preferred_element_type=_f32) + bu2_ref[...]
    so_ref[...] = s
    vo_ref[...] = v_new
    if last:
        x_ref[...] = p_new
    else:
        sn, vn = _norm_pack(s, v_new, None, g_ref[...], b_ref[...])
        z = jnp.zeros((s.shape[0], PK - SDIM - VFD - PP), _f32)
        x_ref[...] = jnp.concatenate([sn, vn, p_new, z], axis=-1)


def _combine(last, pk, ssum, vsum, wu1, bu1, wu2, bu2, g, b):
    kb = functools.partial(_combine_body, last)
    xw = PP if last else PK
    return pl.pallas_call(
        kb,
        grid=(NN // BN,),
        in_specs=[
            pl.BlockSpec((BN, PK), lambda i: (i, 0)),
            pl.BlockSpec((2, BN, 32), lambda i: (0, i, 0)),
            pl.BlockSpec((2, BN, 32), lambda i: (0, i, 0)),
            pl.BlockSpec((SDIM, HID), lambda i: (0, 0)),
            pl.BlockSpec((1, HID), lambda i: (0, 0)),
            pl.BlockSpec((HID, SDIM), lambda i: (0, 0)),
            pl.BlockSpec((1, SDIM), lambda i: (0, 0)),
            pl.BlockSpec((1, SDIM), lambda i: (0, 0)),
            pl.BlockSpec((1, SDIM), lambda i: (0, 0)),
        ],
        out_specs=[
            pl.BlockSpec((BN, SDIM), lambda i: (i, 0)),
            pl.BlockSpec((BN, VFD), lambda i: (i, 0)),
            pl.BlockSpec((BN, xw), lambda i: (i, 0)),
        ],
        out_shape=[
            jax.ShapeDtypeStruct((NN, SDIM), _f32),
            jax.ShapeDtypeStruct((NN, VFD), _f32),
            jax.ShapeDtypeStruct((NN, xw), _f32),
        ],
    )(pk, ssum, vsum, wu1, bu1.reshape(1, HID), wu2, bu2.reshape(1, SDIM),
      g.reshape(1, SDIM), b.reshape(1, SDIM))


def kernel(s, v, p, edge_index_local, d_local, a_local, r_norm_local, e_local,
           edge_index_global, d_global, a_global, r_norm_global, e_global,
           W1, b1, W2, b2, gamma, beta, Wu1, bu1, Wu2, bu2):
    nl = W1.shape[0]
    src = edge_index_global[0].astype(jnp.int32)
    dst = edge_index_global[1].astype(jnp.int32)

    v = v.reshape(NN, VFD)
    p8 = jnp.pad(p, ((0, 0), (0, PP - 3)))

    z1 = jnp.zeros((nl, PK - SDIM - VFD - PP, HID), _f32)
    zv1 = jnp.zeros((nl, VFD + PP, HID), _f32)
    w1s = jnp.concatenate([W1[:, 0:SDIM, :], zv1, z1], axis=1)
    w1d = jnp.concatenate([W1[:, SDIM:2 * SDIM, :], zv1, z1], axis=1)
    w1c = W1[:, 2 * SDIM:2 * SDIM + EDIM, :]
    w1da = W1[:, 2 * SDIM + EDIM:, :]
    sd, vd = SDIM, 16
    w2r = jnp.concatenate([
        W2[:, :, 0:sd],
        W2[:, :, sd:sd + vd],
        W2[:, :, sd + vd:sd + 2 * vd],
        W2[:, :, sd + 2 * vd + 1:],
        W2[:, :, sd + 2 * vd:sd + 2 * vd + 1],
        jnp.zeros((nl, HID, 128 - 113), _f32),
    ], axis=-1)
    b2r = jnp.concatenate([
        b2[:, 0:sd], b2[:, sd:sd + vd], b2[:, sd + vd:sd + 2 * vd],
        b2[:, sd + 2 * vd + 1:], b2[:, sd + 2 * vd:sd + 2 * vd + 1],
        jnp.zeros((nl, 128 - 113), _f32),
    ], axis=-1)

    zv = jnp.zeros((NN, 32), _f32)

    e = e_global
    da1 = jnp.concatenate(
        [d_global[:, None], a_global[:, None], r_norm_global,
         jnp.zeros((EE, 3), _f32)], axis=-1)

    pk = _pack0(s, v, p8, gamma[0], beta[0])
    for i in range(nl):
        srow, drow = _gather_edges(pk, src, dst)
        sm, vm, e = _edge_mlp(
            i == 0, srow, drow, e, da1,
            w1s[i], w1d[i], w1c[i], w1da[i], b1[i].reshape(1, HID),
            w2r[i], b2r[i].reshape(1, 128))
        ssum = _scatter32(sm, dst, zv)
        vsum = _scatter32(vm, dst, zv)
        last = i == nl - 1
        gi = min(i + 1, nl - 1)
        s, v, pk = _combine(
            last, pk, ssum, vsum,
            Wu1[i], bu1[i], Wu2[i], bu2[i], gamma[gi], beta[gi])

    return (s, v.reshape(NN, 3, 16), e, pk[:, 0:3])


# speedup vs baseline: 34.6530x; 1.4356x over previous
import functools

import jax
import jax.numpy as jnp
from jax import lax
from jax.experimental import pallas as pl
from jax.experimental.pallas import tpu as pltpu
from jax.experimental.pallas import tpu_sc as plsc

NN = 50000
EE = 800000
SDIM = 64
VFD = 48
EDIM = 16
HID = 64
PP = 8
PK = 128

NCORES = 2
NSUB = 16
NWORK = NCORES * NSUB

BN = 1000
BE = 2000
GC = 1000
SCC = 400

EPW = EE // NWORK
GITERS = EPW // GC
EPT = EE // NSUB
SITERS = EPT // SCC
NPT = NN // NSUB

_f32 = jnp.float32


def _norm_pack(s, v, p8, gamma_i, beta_i):
    sn = gamma_i * (s - jnp.mean(s, axis=-1, keepdims=True)) / jnp.sqrt(
        jnp.mean((s - jnp.mean(s, axis=-1, keepdims=True)) ** 2,
                 axis=-1, keepdims=True) + 1e-6) + beta_i
    vn = v / jnp.sqrt(jnp.sum(v * v, axis=-1, keepdims=True) / 16.0 + 1e-6)
    return sn, vn


def _pack0_body(s_ref, v_ref, p_ref, g_ref, b_ref, pk_ref):
    sn, vn = _norm_pack(s_ref[...], v_ref[...], None, g_ref[...], b_ref[...])
    z = jnp.zeros((sn.shape[0], PK - SDIM - VFD - PP), _f32)
    pk_ref[...] = jnp.concatenate([sn, vn, p_ref[...], z], axis=-1)


def _pack0(s, v, p8, gamma_i, beta_i):
    return pl.pallas_call(
        _pack0_body,
        grid=(NN // BN,),
        in_specs=[
            pl.BlockSpec((BN, SDIM), lambda i: (i, 0)),
            pl.BlockSpec((BN, VFD), lambda i: (i, 0)),
            pl.BlockSpec((BN, PP), lambda i: (i, 0)),
            pl.BlockSpec((1, SDIM), lambda i: (0, 0)),
            pl.BlockSpec((1, SDIM), lambda i: (0, 0)),
        ],
        out_specs=pl.BlockSpec((BN, PK), lambda i: (i, 0)),
        out_shape=jax.ShapeDtypeStruct((NN, PK), _f32),
    )(s, v, p8, gamma_i.reshape(1, SDIM), beta_i.reshape(1, SDIM))


def _gather_body(pk_hbm, src_hbm, dst_hbm, srow_o, drow_o,
                 isrc_v, idst_v, brow, sem):
    wid = lax.axis_index("s") * NCORES + lax.axis_index("c")
    base0 = wid * EPW

    def chunk(ci):
        base = base0 + ci * GC
        pltpu.sync_copy(src_hbm.at[pl.ds(base, GC)], isrc_v)
        pltpu.sync_copy(dst_hbm.at[pl.ds(base, GC)], idst_v)
        pltpu.async_copy(pk_hbm.at[isrc_v], brow, sem).wait()
        pltpu.sync_copy(brow, srow_o.at[pl.ds(base, GC)])
        pltpu.async_copy(pk_hbm.at[idst_v], brow, sem).wait()
        pltpu.sync_copy(brow, drow_o.at[pl.ds(base, GC)])

    pl.loop(0, GITERS)(chunk)


def _gather_edges(pk, src, dst):
    mesh = plsc.VectorSubcoreMesh(core_axis_name="c", subcore_axis_name="s")
    k = pl.kernel(
        _gather_body,
        out_type=[
            jax.ShapeDtypeStruct((EE, PK), _f32),
            jax.ShapeDtypeStruct((EE, PK), _f32),
        ],
        mesh=mesh,
        scratch_types=[
            pltpu.VMEM((GC,), jnp.int32),
            pltpu.VMEM((GC,), jnp.int32),
            pltpu.VMEM((GC, PK), _f32),
            pltpu.SemaphoreType.DMA,
        ],
    )
    return k(pk, src, dst)


def _edge_body(first, srow_ref, drow_ref, e_ref, da_ref,
               w1s_ref, w1d_ref, w1c_ref, w1x8_ref, b1_ref,
               w2_ref, b2_ref, t16_ref, k848_ref, k83_ref, sh8_ref,
               msg_o, en_o):
    srow = srow_ref[...]
    drow = drow_ref[...]
    ones81 = jnp.ones((PP, 1), _f32)
    ones18 = jnp.ones((1, PP), _f32)

    h = jnp.dot(srow, w1s_ref[...], preferred_element_type=_f32)
    h = h + jnp.dot(drow, w1d_ref[...], preferred_element_type=_f32)
    h = h + jnp.dot(e_ref[...], w1c_ref[...], preferred_element_type=_f32)
    if first:
        x1 = da_ref[...]
        h = h + jnp.dot(x1, w1x8_ref[...], preferred_element_type=_f32)
        rn8 = jnp.dot(x1, sh8_ref[...], preferred_element_type=_f32)
    else:
        psrc = srow[:, 112:120]
        pdst = drow[:, 112:120]
        r = pdst - psrc
        a = jnp.dot(pdst * psrc, ones81, preferred_element_type=_f32)
        d2 = jnp.dot(r * r, ones81, preferred_element_type=_f32)
        d = jnp.sqrt(jnp.maximum(d2, 1e-6))
        inv = 1.0 / (1.0 + d)
        rn8 = r * jnp.dot(inv, ones18, preferred_element_type=_f32)
        da2 = jnp.concatenate([d, a], axis=-1)
        h = h + jnp.dot(da2, w1x8_ref[...][0:2], preferred_element_type=_f32)
    h = h + b1_ref[...]
    h = h * jax.nn.sigmoid(h)
    out = jnp.dot(h, w2_ref[...], preferred_element_type=_f32) + b2_ref[...]

    s_msg = out[:, 0:SDIM]
    w_vv = out[:, SDIM:SDIM + 16]
    w_vs = out[:, SDIM + 16:SDIM + 32]
    e_new = out[:, SDIM + 32:SDIM + 48]
    w_p = out[:, SDIM + 48:SDIM + 49]

    wvv48 = jnp.dot(w_vv, t16_ref[...], preferred_element_type=_f32)
    wvs48 = jnp.dot(w_vs, t16_ref[...], preferred_element_type=_f32)
    rn48 = jnp.dot(rn8, k848_ref[...], preferred_element_type=_f32)
    rn3 = jnp.dot(rn8, k83_ref[...], preferred_element_type=_f32)
    vsrc = srow[:, SDIM:SDIM + VFD]
    vm = vsrc * wvv48 + rn48 * wvs48
    tnh3 = jnp.dot(jnp.tanh(w_p), jnp.ones((1, 3), _f32),
                   preferred_element_type=_f32)
    tp = rn3 * tnh3
    ones = jnp.ones((tp.shape[0], 1), _f32)
    z4 = jnp.zeros((tp.shape[0], 4), _f32)
    z8 = jnp.zeros((tp.shape[0], 8), _f32)
    msg_o[...] = jnp.concatenate(
        [s_msg, vm[:, 0:24], tp, ones, z4, vm[:, 24:48], z8], axis=-1)
    en_o[...] = e_new


def _edge_mlp(first, srow, drow, e, da,
              w1s, w1d, w1c, w1x8, b1, w2, b2, t16, k848, k83, sh8):
    kb = functools.partial(_edge_body, first)
    return pl.pallas_call(
        kb,
        grid=(EE // BE,),
        in_specs=[
            pl.BlockSpec((BE, PK), lambda i: (i, 0)),
            pl.BlockSpec((BE, PK), lambda i: (i, 0)),
            pl.BlockSpec((BE, EDIM), lambda i: (i, 0)),
            pl.BlockSpec((BE, PP), lambda i: (i, 0)),
            pl.BlockSpec((PK, HID), lambda i: (0, 0)),
            pl.BlockSpec((PK, HID), lambda i: (0, 0)),
            pl.BlockSpec((EDIM, HID), lambda i: (0, 0)),
            pl.BlockSpec((PP, HID), lambda i: (0, 0)),
            pl.BlockSpec((1, HID), lambda i: (0, 0)),
            pl.BlockSpec((HID, 128), lambda i: (0, 0)),
            pl.BlockSpec((1, 128), lambda i: (0, 0)),
            pl.BlockSpec((16, VFD), lambda i: (0, 0)),
            pl.BlockSpec((PP, VFD), lambda i: (0, 0)),
            pl.BlockSpec((PP, 3), lambda i: (0, 0)),
            pl.BlockSpec((PP, PP), lambda i: (0, 0)),
        ],
        out_specs=[
            pl.BlockSpec((BE, PK), lambda i: (i, 0)),
            pl.BlockSpec((BE, EDIM), lambda i: (i, 0)),
        ],
        out_shape=[
            jax.ShapeDtypeStruct((EE, PK), _f32),
            jax.ShapeDtypeStruct((EE, EDIM), _f32),
        ],
    )(srow, drow, e, da, w1s, w1d, w1c, w1x8, b1, w2, b2, t16, k848, k83, sh8)


def _scat_body(co, msg_hbm, dst_hbm, z_hbm, out_hbm,
               idx_v, msg_v, acc, sem):
    cid = lax.axis_index("c")
    sid = lax.axis_index("s")
    r0 = sid * NPT
    pltpu.sync_copy(z_hbm.at[pl.ds(r0, NPT)], acc.at[pl.ds(r0, NPT)])
    plsc.subcore_barrier()

    def chunk(ci):
        base = sid * EPT + ci * SCC
        pltpu.sync_copy(dst_hbm.at[pl.ds(base, SCC)], idx_v)
        pltpu.sync_copy(
            msg_hbm.at[pl.ds(base, SCC), pl.ds(co + cid * 32, 32)], msg_v)
        pltpu.sync_copy(msg_v, acc.at[idx_v], add=True)

    pl.loop(0, SITERS)(chunk)
    plsc.subcore_barrier()
    pltpu.sync_copy(acc.at[pl.ds(r0, NPT)], out_hbm.at[cid, pl.ds(r0, NPT)])


def _scatter32(msg, dst, z, co):
    mesh = plsc.VectorSubcoreMesh(core_axis_name="c", subcore_axis_name="s")
    k = pl.kernel(
        functools.partial(_scat_body, co),
        out_type=jax.ShapeDtypeStruct((2, NN, 32), _f32),
        mesh=mesh,
        compiler_params=pltpu.CompilerParams(use_tc_tiling_on_sc=False),
        scratch_types=[
            pltpu.VMEM((SCC,), jnp.int32),
            pltpu.VMEM((SCC, 32), _f32),
            pltpu.VMEM_SHARED((NN, 32), _f32),
            pltpu.SemaphoreType.DMA,
        ],
    )
    return k(msg, dst, z)


def _combine_body(last, pk_ref, ss_ref, vs_ref,
                  wu1_ref, bu1_ref, wu2_ref, bu2_ref, g_ref, b_ref,
                  so_ref, vo_ref, x_ref):
    pk = pk_ref[...]
    vs0 = vs_ref[0]
    vs1 = vs_ref[1]
    cnt = jnp.maximum(vs0[:, 27:28], 1.0)
    vsum = jnp.concatenate([vs0[:, 0:24], vs1[:, 0:24]], axis=-1)
    v_new = pk[:, SDIM:SDIM + VFD] + vsum / cnt
    p_new = jnp.concatenate(
        [pk[:, 112:115] + vs0[:, 24:27] / cnt,
         jnp.zeros((pk.shape[0], PP - 3), _f32)], axis=-1)
    s = pk[:, 0:SDIM] + jnp.concatenate([ss_ref[0], ss_ref[1]], axis=-1)
    if not last:
        h = jnp.dot(s, wu1_ref[...], preferred_element_type=_f32) + bu1_ref[...]
        h = h * jax.nn.sigmoid(h)
        s = s + jnp.dot(h, wu2_ref[...], preferred_element_type=_f32) + bu2_ref[...]
    so_ref[...] = s
    vo_ref[...] = v_new
    if last:
        x_ref[...] = p_new
    else:
        sn, vn = _norm_pack(s, v_new, None, g_ref[...], b_ref[...])
        z = jnp.zeros((s.shape[0], PK - SDIM - VFD - PP), _f32)
        x_ref[...] = jnp.concatenate([sn, vn, p_new, z], axis=-1)


def _combine(last, pk, ssum, vsum, wu1, bu1, wu2, bu2, g, b):
    kb = functools.partial(_combine_body, last)
    xw = PP if last else PK
    return pl.pallas_call(
        kb,
        grid=(NN // BN,),
        in_specs=[
            pl.BlockSpec((BN, PK), lambda i: (i, 0)),
            pl.BlockSpec((2, BN, 32), lambda i: (0, i, 0)),
            pl.BlockSpec((2, BN, 32), lambda i: (0, i, 0)),
            pl.BlockSpec((SDIM, HID), lambda i: (0, 0)),
            pl.BlockSpec((1, HID), lambda i: (0, 0)),
            pl.BlockSpec((HID, SDIM), lambda i: (0, 0)),
            pl.BlockSpec((1, SDIM), lambda i: (0, 0)),
            pl.BlockSpec((1, SDIM), lambda i: (0, 0)),
            pl.BlockSpec((1, SDIM), lambda i: (0, 0)),
        ],
        out_specs=[
            pl.BlockSpec((BN, SDIM), lambda i: (i, 0)),
            pl.BlockSpec((BN, VFD), lambda i: (i, 0)),
            pl.BlockSpec((BN, xw), lambda i: (i, 0)),
        ],
        out_shape=[
            jax.ShapeDtypeStruct((NN, SDIM), _f32),
            jax.ShapeDtypeStruct((NN, VFD), _f32),
            jax.ShapeDtypeStruct((NN, xw), _f32),
        ],
    )(pk, ssum, vsum, wu1, bu1.reshape(1, HID), wu2, bu2.reshape(1, SDIM),
      g.reshape(1, SDIM), b.reshape(1, SDIM))


def kernel(s, v, p, edge_index_local, d_local, a_local, r_norm_local, e_local,
           edge_index_global, d_global, a_global, r_norm_global, e_global,
           W1, b1, W2, b2, gamma, beta, Wu1, bu1, Wu2, bu2):
    nl = W1.shape[0]
    src = edge_index_global[0].astype(jnp.int32)
    dst = edge_index_global[1].astype(jnp.int32)

    v = v.reshape(NN, VFD)
    p8 = jnp.pad(p, ((0, 0), (0, PP - 3)))

    z1 = jnp.zeros((nl, PK - SDIM - VFD - PP, HID), _f32)
    zv1 = jnp.zeros((nl, VFD + PP, HID), _f32)
    w1s = jnp.concatenate([W1[:, 0:SDIM, :], zv1, z1], axis=1)
    w1d = jnp.concatenate([W1[:, SDIM:2 * SDIM, :], zv1, z1], axis=1)
    w1c = W1[:, 2 * SDIM:2 * SDIM + EDIM, :]
    w1x8 = jnp.concatenate(
        [W1[:, 2 * SDIM + EDIM:, :], jnp.zeros((nl, PP - 2, HID), _f32)],
        axis=1)
    t16 = jnp.tile(jnp.eye(16, dtype=_f32), (1, 3))
    k848 = jnp.concatenate(
        [jnp.kron(jnp.eye(3, dtype=_f32), jnp.ones((1, 16), _f32)),
         jnp.zeros((PP - 3, VFD), _f32)], axis=0)
    k83 = jnp.concatenate(
        [jnp.eye(3, dtype=_f32), jnp.zeros((PP - 3, 3), _f32)], axis=0)
    sh8 = jnp.pad(jnp.eye(3, dtype=_f32), ((2, PP - 5), (0, PP - 3)))
    sd, vd = SDIM, 16
    w2r = jnp.concatenate([
        W2[:, :, 0:sd],
        W2[:, :, sd:sd + vd],
        W2[:, :, sd + vd:sd + 2 * vd],
        W2[:, :, sd + 2 * vd + 1:],
        W2[:, :, sd + 2 * vd:sd + 2 * vd + 1],
        jnp.zeros((nl, HID, 128 - 113), _f32),
    ], axis=-1)
    b2r = jnp.concatenate([
        b2[:, 0:sd], b2[:, sd:sd + vd], b2[:, sd + vd:sd + 2 * vd],
        b2[:, sd + 2 * vd + 1:], b2[:, sd + 2 * vd:sd + 2 * vd + 1],
        jnp.zeros((nl, 128 - 113), _f32),
    ], axis=-1)

    zv = jnp.zeros((NN, 32), _f32)

    e = e_global
    da1 = jnp.concatenate(
        [d_global[:, None], a_global[:, None], r_norm_global,
         jnp.zeros((EE, 3), _f32)], axis=-1)

    pk = _pack0(s, v, p8, gamma[0], beta[0])
    for i in range(nl):
        srow, drow = _gather_edges(pk, src, dst)
        msg, e = _edge_mlp(
            i == 0, srow, drow, e, da1,
            w1s[i], w1d[i], w1c[i], w1x8[i], b1[i].reshape(1, HID),
            w2r[i], b2r[i].reshape(1, 128), t16, k848, k83, sh8)
        ssum = _scatter32(msg, dst, zv, 0)
        vsum = _scatter32(msg, dst, zv, SDIM)
        last = i == nl - 1
        gi = min(i + 1, nl - 1)
        s, v, pk = _combine(
            last, pk, ssum, vsum,
            Wu1[i], bu1[i], Wu2[i], bu2[i], gamma[gi], beta[gi])

    return (s, v.reshape(NN, 3, 16), e, pk[:, 0:3])


# speedup vs baseline: 36.9139x; 1.0652x over previous
import functools

import jax
import jax.numpy as jnp
from jax import lax
from jax.experimental import pallas as pl
from jax.experimental.pallas import tpu as pltpu
from jax.experimental.pallas import tpu_sc as plsc

NN = 50000
EE = 800000
SDIM = 64
VFD = 48
EDIM = 16
HID = 64
PP = 8
PK = 128

NCORES = 2
NSUB = 16
NWORK = NCORES * NSUB

BN = 1000
BE = 2000
GC = 1000
SCC = 400

EPW = EE // NWORK
GITERS = EPW // GC
EPT = EE // NSUB
SITERS = EPT // SCC
NPT = NN // NSUB

_f32 = jnp.float32


def _norm_pack(s, v, p8, gamma_i, beta_i):
    sn = gamma_i * (s - jnp.mean(s, axis=-1, keepdims=True)) / jnp.sqrt(
        jnp.mean((s - jnp.mean(s, axis=-1, keepdims=True)) ** 2,
                 axis=-1, keepdims=True) + 1e-6) + beta_i
    vn = v / jnp.sqrt(jnp.sum(v * v, axis=-1, keepdims=True) / 16.0 + 1e-6)
    return sn, vn


def _pack0_body(s_ref, v_ref, p_ref, g_ref, b_ref, pk_ref):
    sn, vn = _norm_pack(s_ref[...], v_ref[...], None, g_ref[...], b_ref[...])
    z = jnp.zeros((sn.shape[0], PK - SDIM - VFD - PP), _f32)
    pk_ref[...] = jnp.concatenate([sn, vn, p_ref[...], z], axis=-1)


def _pack0(s, v, p8, gamma_i, beta_i):
    return pl.pallas_call(
        _pack0_body,
        grid=(NN // BN,),
        in_specs=[
            pl.BlockSpec((BN, SDIM), lambda i: (i, 0)),
            pl.BlockSpec((BN, VFD), lambda i: (i, 0)),
            pl.BlockSpec((BN, PP), lambda i: (i, 0)),
            pl.BlockSpec((1, SDIM), lambda i: (0, 0)),
            pl.BlockSpec((1, SDIM), lambda i: (0, 0)),
        ],
        out_specs=pl.BlockSpec((BN, PK), lambda i: (i, 0)),
        out_shape=jax.ShapeDtypeStruct((NN, PK), _f32),
    )(s, v, p8, gamma_i.reshape(1, SDIM), beta_i.reshape(1, SDIM))


def _gather_body(pk_hbm, src_hbm, dst_hbm, srow_o, drow_o,
                 isrc_v, idst_v, brow, sem):
    wid = lax.axis_index("s") * NCORES + lax.axis_index("c")
    base0 = wid * EPW

    def chunk(ci):
        base = base0 + ci * GC
        pltpu.sync_copy(src_hbm.at[pl.ds(base, GC)], isrc_v)
        pltpu.sync_copy(dst_hbm.at[pl.ds(base, GC)], idst_v)
        pltpu.async_copy(pk_hbm.at[isrc_v], brow, sem).wait()
        pltpu.sync_copy(brow, srow_o.at[pl.ds(base, GC)])
        pltpu.async_copy(pk_hbm.at[idst_v], brow, sem).wait()
        pltpu.sync_copy(brow, drow_o.at[pl.ds(base, GC)])

    pl.loop(0, GITERS)(chunk)


def _gather_edges(pk, src, dst):
    mesh = plsc.VectorSubcoreMesh(core_axis_name="c", subcore_axis_name="s")
    k = pl.kernel(
        _gather_body,
        out_type=[
            jax.ShapeDtypeStruct((EE, PK), _f32),
            jax.ShapeDtypeStruct((EE, PK), _f32),
        ],
        mesh=mesh,
        scratch_types=[
            pltpu.VMEM((GC,), jnp.int32),
            pltpu.VMEM((GC,), jnp.int32),
            pltpu.VMEM((GC, PK), _f32),
            pltpu.SemaphoreType.DMA,
        ],
    )
    return k(pk, src, dst)


def _edge_body(first, srow_ref, drow_ref, e_ref, da_ref,
               w1s_ref, w1d_ref, w1c_ref, w1x8_ref, w1a8_ref, w1d8_ref,
               b1_ref, w2_ref, b2_ref, t16_ref, k848_ref, sh8_ref,
               msg_o, en_o):
    srow = srow_ref[...]
    drow = drow_ref[...]

    h = jnp.dot(srow, w1s_ref[...], preferred_element_type=_f32)
    h = h + jnp.dot(drow, w1d_ref[...], preferred_element_type=_f32)
    h = h + jnp.dot(e_ref[...], w1c_ref[...], preferred_element_type=_f32)
    if first:
        x1 = da_ref[...]
        h = h + jnp.dot(x1, w1x8_ref[...], preferred_element_type=_f32)
        rn8 = jnp.dot(x1, sh8_ref[...], preferred_element_type=_f32)
    else:
        psrc = srow[:, 112:120]
        pdst = drow[:, 112:120]
        r = pdst - psrc
        h = h + jnp.dot(pdst * psrc, w1a8_ref[...],
                        preferred_element_type=_f32)
        d2_8 = jnp.dot(r * r, jnp.ones((PP, PP), _f32),
                       preferred_element_type=_f32)
        d8 = jnp.sqrt(jnp.maximum(d2_8, 1e-6))
        h = h + jnp.dot(d8, w1d8_ref[...], preferred_element_type=_f32)
        rn8 = r / (1.0 + d8)
    h = h + b1_ref[...]
    h = h * jax.nn.sigmoid(h)
    out = jnp.dot(h, w2_ref[...], preferred_element_type=_f32) + b2_ref[...]

    s_msg = out[:, 0:SDIM]
    w_vv = out[:, SDIM:SDIM + 16]
    w_vs = out[:, SDIM + 16:SDIM + 32]
    e_new = out[:, SDIM + 32:SDIM + 48]
    w_p8 = out[:, 112:120]

    wvv48 = jnp.dot(w_vv, t16_ref[...], preferred_element_type=_f32)
    wvs48 = jnp.dot(w_vs, t16_ref[...], preferred_element_type=_f32)
    rn48 = jnp.dot(rn8, k848_ref[...], preferred_element_type=_f32)
    vsrc = srow[:, SDIM:SDIM + VFD]
    vm = vsrc * wvv48 + rn48 * wvs48
    tp8 = rn8 * jnp.tanh(w_p8)
    ones = jnp.ones((vm.shape[0], 1), _f32)
    z4 = jnp.zeros((vm.shape[0], 4), _f32)
    z8 = jnp.zeros((vm.shape[0], 8), _f32)
    msg_o[...] = jnp.concatenate(
        [s_msg, vm[:, 0:24], tp8[:, 0:3], ones, z4, vm[:, 24:48], z8],
        axis=-1)
    en_o[...] = e_new


def _edge_mlp(first, srow, drow, e, da,
              w1s, w1d, w1c, w1x8, w1a8, w1d8, b1, w2, b2, t16, k848, sh8):
    kb = functools.partial(_edge_body, first)
    return pl.pallas_call(
        kb,
        grid=(EE // BE,),
        in_specs=[
            pl.BlockSpec((BE, PK), lambda i: (i, 0)),
            pl.BlockSpec((BE, PK), lambda i: (i, 0)),
            pl.BlockSpec((BE, EDIM), lambda i: (i, 0)),
            pl.BlockSpec((BE, PP), lambda i: (i, 0)),
            pl.BlockSpec((PK, HID), lambda i: (0, 0)),
            pl.BlockSpec((PK, HID), lambda i: (0, 0)),
            pl.BlockSpec((EDIM, HID), lambda i: (0, 0)),
            pl.BlockSpec((PP, HID), lambda i: (0, 0)),
            pl.BlockSpec((PP, HID), lambda i: (0, 0)),
            pl.BlockSpec((PP, HID), lambda i: (0, 0)),
            pl.BlockSpec((1, HID), lambda i: (0, 0)),
            pl.BlockSpec((HID, 128), lambda i: (0, 0)),
            pl.BlockSpec((1, 128), lambda i: (0, 0)),
            pl.BlockSpec((16, VFD), lambda i: (0, 0)),
            pl.BlockSpec((PP, VFD), lambda i: (0, 0)),
            pl.BlockSpec((PP, PP), lambda i: (0, 0)),
        ],
        out_specs=[
            pl.BlockSpec((BE, PK), lambda i: (i, 0)),
            pl.BlockSpec((BE, EDIM), lambda i: (i, 0)),
        ],
        out_shape=[
            jax.ShapeDtypeStruct((EE, PK), _f32),
            jax.ShapeDtypeStruct((EE, EDIM), _f32),
        ],
    )(srow, drow, e, da, w1s, w1d, w1c, w1x8, w1a8, w1d8, b1, w2, b2,
      t16, k848, sh8)


def _scat_body(co, msg_hbm, dst_hbm, z_hbm, out_hbm,
               idx_v, msg_v, acc, sem):
    cid = lax.axis_index("c")
    sid = lax.axis_index("s")
    r0 = sid * NPT
    pltpu.sync_copy(z_hbm.at[pl.ds(r0, NPT)], acc.at[pl.ds(r0, NPT)])
    plsc.subcore_barrier()

    def chunk(ci):
        base = sid * EPT + ci * SCC
        pltpu.sync_copy(dst_hbm.at[pl.ds(base, SCC)], idx_v)
        pltpu.sync_copy(
            msg_hbm.at[pl.ds(base, SCC), pl.ds(co + cid * 32, 32)], msg_v)
        pltpu.sync_copy(msg_v, acc.at[idx_v], add=True)

    pl.loop(0, SITERS)(chunk)
    plsc.subcore_barrier()
    pltpu.sync_copy(acc.at[pl.ds(r0, NPT)], out_hbm.at[cid, pl.ds(r0, NPT)])


def _scatter32(msg, dst, z, co):
    mesh = plsc.VectorSubcoreMesh(core_axis_name="c", subcore_axis_name="s")
    k = pl.kernel(
        functools.partial(_scat_body, co),
        out_type=jax.ShapeDtypeStruct((2, NN, 32), _f32),
        mesh=mesh,
        compiler_params=pltpu.CompilerParams(use_tc_tiling_on_sc=False),
        scratch_types=[
            pltpu.VMEM((SCC,), jnp.int32),
            pltpu.VMEM((SCC, 32), _f32),
            pltpu.VMEM_SHARED((NN, 32), _f32),
            pltpu.SemaphoreType.DMA,
        ],
    )
    return k(msg, dst, z)


def _combine_body(last, pk_ref, ss_ref, vs_ref,
                  wu1_ref, bu1_ref, wu2_ref, bu2_ref, g_ref, b_ref,
                  so_ref, vo_ref, x_ref):
    pk = pk_ref[...]
    vs0 = vs_ref[0]
    vs1 = vs_ref[1]
    cnt = jnp.maximum(vs0[:, 27:28], 1.0)
    vsum = jnp.concatenate([vs0[:, 0:24], vs1[:, 0:24]], axis=-1)
    v_new = pk[:, SDIM:SDIM + VFD] + vsum / cnt
    p_new = jnp.concatenate(
        [pk[:, 112:115] + vs0[:, 24:27] / cnt,
         jnp.zeros((pk.shape[0], PP - 3), _f32)], axis=-1)
    s = pk[:, 0:SDIM] + jnp.concatenate([ss_ref[0], ss_ref[1]], axis=-1)
    if not last:
        h = jnp.dot(s, wu1_ref[...], preferred_element_type=_f32) + bu1_ref[...]
        h = h * jax.nn.sigmoid(h)
        s = s + jnp.dot(h, wu2_ref[...], preferred_element_type=_f32) + bu2_ref[...]
    so_ref[...] = s
    vo_ref[...] = v_new
    if last:
        x_ref[...] = p_new
    else:
        sn, vn = _norm_pack(s, v_new, None, g_ref[...], b_ref[...])
        z = jnp.zeros((s.shape[0], PK - SDIM - VFD - PP), _f32)
        x_ref[...] = jnp.concatenate([sn, vn, p_new, z], axis=-1)


def _combine(last, pk, ssum, vsum, wu1, bu1, wu2, bu2, g, b):
    kb = functools.partial(_combine_body, last)
    xw = PP if last else PK
    return pl.pallas_call(
        kb,
        grid=(NN // BN,),
        in_specs=[
            pl.BlockSpec((BN, PK), lambda i: (i, 0)),
            pl.BlockSpec((2, BN, 32), lambda i: (0, i, 0)),
            pl.BlockSpec((2, BN, 32), lambda i: (0, i, 0)),
            pl.BlockSpec((SDIM, HID), lambda i: (0, 0)),
            pl.BlockSpec((1, HID), lambda i: (0, 0)),
            pl.BlockSpec((HID, SDIM), lambda i: (0, 0)),
            pl.BlockSpec((1, SDIM), lambda i: (0, 0)),
            pl.BlockSpec((1, SDIM), lambda i: (0, 0)),
            pl.BlockSpec((1, SDIM), lambda i: (0, 0)),
        ],
        out_specs=[
            pl.BlockSpec((BN, SDIM), lambda i: (i, 0)),
            pl.BlockSpec((BN, VFD), lambda i: (i, 0)),
            pl.BlockSpec((BN, xw), lambda i: (i, 0)),
        ],
        out_shape=[
            jax.ShapeDtypeStruct((NN, SDIM), _f32),
            jax.ShapeDtypeStruct((NN, VFD), _f32),
            jax.ShapeDtypeStruct((NN, xw), _f32),
        ],
    )(pk, ssum, vsum, wu1, bu1.reshape(1, HID), wu2, bu2.reshape(1, SDIM),
      g.reshape(1, SDIM), b.reshape(1, SDIM))


def kernel(s, v, p, edge_index_local, d_local, a_local, r_norm_local, e_local,
           edge_index_global, d_global, a_global, r_norm_global, e_global,
           W1, b1, W2, b2, gamma, beta, Wu1, bu1, Wu2, bu2):
    nl = W1.shape[0]
    src = edge_index_global[0].astype(jnp.int32)
    dst = edge_index_global[1].astype(jnp.int32)

    v = v.reshape(NN, VFD)
    p8 = jnp.pad(p, ((0, 0), (0, PP - 3)))

    z1 = jnp.zeros((nl, PK - SDIM - VFD - PP, HID), _f32)
    zv1 = jnp.zeros((nl, VFD + PP, HID), _f32)
    w1s = jnp.concatenate([W1[:, 0:SDIM, :], zv1, z1], axis=1)
    w1d = jnp.concatenate([W1[:, SDIM:2 * SDIM, :], zv1, z1], axis=1)
    w1c = W1[:, 2 * SDIM:2 * SDIM + EDIM, :]
    w1x8 = jnp.concatenate(
        [W1[:, 2 * SDIM + EDIM:, :], jnp.zeros((nl, PP - 2, HID), _f32)],
        axis=1)
    t16 = jnp.tile(jnp.eye(16, dtype=_f32), (1, 3))
    k848 = jnp.concatenate(
        [jnp.kron(jnp.eye(3, dtype=_f32), jnp.ones((1, 16), _f32)),
         jnp.zeros((PP - 3, VFD), _f32)], axis=0)
    sh8 = jnp.pad(jnp.eye(3, dtype=_f32), ((2, PP - 5), (0, PP - 3)))
    d_row = W1[:, 2 * SDIM + EDIM, :]
    a_row = W1[:, 2 * SDIM + EDIM + 1, :]
    w1a8 = jnp.tile(a_row[:, None, :], (1, PP, 1))
    w1d8 = jnp.tile(d_row[:, None, :], (1, PP, 1)) / 8.0
    sd, vd = SDIM, 16
    wp = W2[:, :, sd + 2 * vd:sd + 2 * vd + 1]
    w2r = jnp.concatenate([
        W2[:, :, 0:sd],
        W2[:, :, sd:sd + vd],
        W2[:, :, sd + vd:sd + 2 * vd],
        W2[:, :, sd + 2 * vd + 1:],
        jnp.tile(wp, (1, 1, PP)),
        jnp.zeros((nl, HID, 8), _f32),
    ], axis=-1)
    bp = b2[:, sd + 2 * vd:sd + 2 * vd + 1]
    b2r = jnp.concatenate([
        b2[:, 0:sd], b2[:, sd:sd + vd], b2[:, sd + vd:sd + 2 * vd],
        b2[:, sd + 2 * vd + 1:], jnp.tile(bp, (1, PP)),
        jnp.zeros((nl, 8), _f32),
    ], axis=-1)

    zv = jnp.zeros((NN, 32), _f32)

    e = e_global
    da1 = jnp.concatenate(
        [d_global[:, None], a_global[:, None], r_norm_global,
         jnp.zeros((EE, 3), _f32)], axis=-1)

    pk = _pack0(s, v, p8, gamma[0], beta[0])
    for i in range(nl):
        srow, drow = _gather_edges(pk, src, dst)
        msg, e = _edge_mlp(
            i == 0, srow, drow, e, da1,
            w1s[i], w1d[i], w1c[i], w1x8[i], w1a8[i], w1d8[i],
            b1[i].reshape(1, HID),
            w2r[i], b2r[i].reshape(1, 128), t16, k848, sh8)
        ssum = _scatter32(msg, dst, zv, 0)
        vsum = _scatter32(msg, dst, zv, SDIM)
        last = i == nl - 1
        gi = min(i + 1, nl - 1)
        s, v, pk = _combine(
            last, pk, ssum, vsum,
            Wu1[i], bu1[i], Wu2[i], bu2[i], gamma[gi], beta[gi])

    return (s, v.reshape(NN, 3, 16), e, pk[:, 0:3])


# speedup vs baseline: 37.1342x; 1.0060x over previous
import functools

import jax
import jax.numpy as jnp
from jax import lax
from jax.experimental import pallas as pl
from jax.experimental.pallas import tpu as pltpu
from jax.experimental.pallas import tpu_sc as plsc

NN = 50000
EE = 800000
SDIM = 64
VFD = 48
EDIM = 16
HID = 64
PP = 8
PK = 128

NCORES = 2
NSUB = 16
NWORK = NCORES * NSUB

BN = 1000
BE = 4000
GC = 1000
SCC = 400

EPW = EE // NWORK
GITERS = EPW // GC
EPT = EE // NSUB
SITERS = EPT // SCC
NPT = NN // NSUB

_f32 = jnp.float32


def _norm_pack(s, v, p8, gamma_i, beta_i):
    sn = gamma_i * (s - jnp.mean(s, axis=-1, keepdims=True)) / jnp.sqrt(
        jnp.mean((s - jnp.mean(s, axis=-1, keepdims=True)) ** 2,
                 axis=-1, keepdims=True) + 1e-6) + beta_i
    vn = v / jnp.sqrt(jnp.sum(v * v, axis=-1, keepdims=True) / 16.0 + 1e-6)
    return sn, vn


def _pack0_body(s_ref, v_ref, p_ref, g_ref, b_ref, pk_ref):
    sn, vn = _norm_pack(s_ref[...], v_ref[...], None, g_ref[...], b_ref[...])
    z = jnp.zeros((sn.shape[0], PK - SDIM - VFD - PP), _f32)
    pk_ref[...] = jnp.concatenate([sn, vn, p_ref[...], z], axis=-1)


def _pack0(s, v, p8, gamma_i, beta_i):
    return pl.pallas_call(
        _pack0_body,
        grid=(NN // BN,),
        in_specs=[
            pl.BlockSpec((BN, SDIM), lambda i: (i, 0)),
            pl.BlockSpec((BN, VFD), lambda i: (i, 0)),
            pl.BlockSpec((BN, PP), lambda i: (i, 0)),
            pl.BlockSpec((1, SDIM), lambda i: (0, 0)),
            pl.BlockSpec((1, SDIM), lambda i: (0, 0)),
        ],
        out_specs=pl.BlockSpec((BN, PK), lambda i: (i, 0)),
        out_shape=jax.ShapeDtypeStruct((NN, PK), _f32),
    )(s, v, p8, gamma_i.reshape(1, SDIM), beta_i.reshape(1, SDIM))


def _gather_body(pk_hbm, src_hbm, dst_hbm, srow_o, drow_o,
                 isrc_v, idst_v, brow, sem):
    wid = lax.axis_index("s") * NCORES + lax.axis_index("c")
    base0 = wid * EPW

    def chunk(ci):
        base = base0 + ci * GC
        pltpu.sync_copy(src_hbm.at[pl.ds(base, GC)], isrc_v)
        pltpu.sync_copy(dst_hbm.at[pl.ds(base, GC)], idst_v)
        pltpu.async_copy(pk_hbm.at[isrc_v], brow, sem).wait()
        pltpu.sync_copy(brow, srow_o.at[pl.ds(base, GC)])
        pltpu.async_copy(pk_hbm.at[idst_v], brow, sem).wait()
        pltpu.sync_copy(brow, drow_o.at[pl.ds(base, GC)])

    pl.loop(0, GITERS)(chunk)


def _gather_edges(pk, src, dst):
    mesh = plsc.VectorSubcoreMesh(core_axis_name="c", subcore_axis_name="s")
    k = pl.kernel(
        _gather_body,
        out_type=[
            jax.ShapeDtypeStruct((EE, PK), _f32),
            jax.ShapeDtypeStruct((EE, PK), _f32),
        ],
        mesh=mesh,
        scratch_types=[
            pltpu.VMEM((GC,), jnp.int32),
            pltpu.VMEM((GC,), jnp.int32),
            pltpu.VMEM((GC, PK), _f32),
            pltpu.SemaphoreType.DMA,
        ],
    )
    return k(pk, src, dst)


def _edge_body(first, srow_ref, drow_ref, e_ref, da_ref,
               w1s_ref, w1d_ref, w1c_ref, w1x8_ref, w1a8_ref, w1d8_ref,
               b1_ref, w2_ref, b2_ref, t16_ref, k848_ref, sh8_ref,
               msg_o, en_o):
    srow = srow_ref[...]
    drow = drow_ref[...]

    h = jnp.dot(srow, w1s_ref[...], preferred_element_type=_f32)
    h = h + jnp.dot(drow, w1d_ref[...], preferred_element_type=_f32)
    h = h + jnp.dot(e_ref[...], w1c_ref[...], preferred_element_type=_f32)
    if first:
        x1 = da_ref[...]
        h = h + jnp.dot(x1, w1x8_ref[...], preferred_element_type=_f32)
        rn8 = jnp.dot(x1, sh8_ref[...], preferred_element_type=_f32)
    else:
        psrc = srow[:, 112:120]
        pdst = drow[:, 112:120]
        r = pdst - psrc
        h = h + jnp.dot(pdst * psrc, w1a8_ref[...],
                        preferred_element_type=_f32)
        d2_8 = jnp.dot(r * r, jnp.ones((PP, PP), _f32),
                       preferred_element_type=_f32)
        d8 = jnp.sqrt(jnp.maximum(d2_8, 1e-6))
        h = h + jnp.dot(d8, w1d8_ref[...], preferred_element_type=_f32)
        rn8 = r / (1.0 + d8)
    h = h + b1_ref[...]
    h = h * jax.nn.sigmoid(h)
    out = jnp.dot(h, w2_ref[...], preferred_element_type=_f32) + b2_ref[...]

    s_msg = out[:, 0:SDIM]
    w_vv = out[:, SDIM:SDIM + 16]
    w_vs = out[:, SDIM + 16:SDIM + 32]
    e_new = out[:, SDIM + 32:SDIM + 48]
    w_p8 = out[:, 112:120]

    wvv48 = jnp.dot(w_vv, t16_ref[...], preferred_element_type=_f32)
    wvs48 = jnp.dot(w_vs, t16_ref[...], preferred_element_type=_f32)
    rn48 = jnp.dot(rn8, k848_ref[...], preferred_element_type=_f32)
    vsrc = srow[:, SDIM:SDIM + VFD]
    vm = vsrc * wvv48 + rn48 * wvs48
    tp8 = rn8 * jnp.tanh(w_p8)
    ones = jnp.ones((vm.shape[0], 1), _f32)
    z4 = jnp.zeros((vm.shape[0], 4), _f32)
    z8 = jnp.zeros((vm.shape[0], 8), _f32)
    msg_o[...] = jnp.concatenate(
        [s_msg, vm[:, 0:24], tp8[:, 0:3], ones, z4, vm[:, 24:48], z8],
        axis=-1)
    en_o[...] = e_new


def _edge_mlp(first, srow, drow, e, da,
              w1s, w1d, w1c, w1x8, w1a8, w1d8, b1, w2, b2, t16, k848, sh8):
    kb = functools.partial(_edge_body, first)
    return pl.pallas_call(
        kb,
        grid=(EE // BE,),
        in_specs=[
            pl.BlockSpec((BE, PK), lambda i: (i, 0)),
            pl.BlockSpec((BE, PK), lambda i: (i, 0)),
            pl.BlockSpec((BE, EDIM), lambda i: (i, 0)),
            pl.BlockSpec((BE, PP), lambda i: (i, 0)),
            pl.BlockSpec((PK, HID), lambda i: (0, 0)),
            pl.BlockSpec((PK, HID), lambda i: (0, 0)),
            pl.BlockSpec((EDIM, HID), lambda i: (0, 0)),
            pl.BlockSpec((PP, HID), lambda i: (0, 0)),
            pl.BlockSpec((PP, HID), lambda i: (0, 0)),
            pl.BlockSpec((PP, HID), lambda i: (0, 0)),
            pl.BlockSpec((1, HID), lambda i: (0, 0)),
            pl.BlockSpec((HID, 128), lambda i: (0, 0)),
            pl.BlockSpec((1, 128), lambda i: (0, 0)),
            pl.BlockSpec((16, VFD), lambda i: (0, 0)),
            pl.BlockSpec((PP, VFD), lambda i: (0, 0)),
            pl.BlockSpec((PP, PP), lambda i: (0, 0)),
        ],
        out_specs=[
            pl.BlockSpec((BE, PK), lambda i: (i, 0)),
            pl.BlockSpec((BE, EDIM), lambda i: (i, 0)),
        ],
        out_shape=[
            jax.ShapeDtypeStruct((EE, PK), _f32),
            jax.ShapeDtypeStruct((EE, EDIM), _f32),
        ],
    )(srow, drow, e, da, w1s, w1d, w1c, w1x8, w1a8, w1d8, b1, w2, b2,
      t16, k848, sh8)


def _scat_body(co, msg_hbm, dst_hbm, z_hbm, out_hbm,
               idx_v, msg_v, acc, sem):
    cid = lax.axis_index("c")
    sid = lax.axis_index("s")
    r0 = sid * NPT
    pltpu.sync_copy(z_hbm.at[pl.ds(r0, NPT)], acc.at[pl.ds(r0, NPT)])
    plsc.subcore_barrier()

    def chunk(ci):
        base = sid * EPT + ci * SCC
        pltpu.sync_copy(dst_hbm.at[pl.ds(base, SCC)], idx_v)
        pltpu.sync_copy(
            msg_hbm.at[pl.ds(base, SCC), pl.ds(co + cid * 32, 32)], msg_v)
        pltpu.sync_copy(msg_v, acc.at[idx_v], add=True)

    pl.loop(0, SITERS)(chunk)
    plsc.subcore_barrier()
    pltpu.sync_copy(acc.at[pl.ds(r0, NPT)], out_hbm.at[cid, pl.ds(r0, NPT)])


def _scatter32(msg, dst, z, co):
    mesh = plsc.VectorSubcoreMesh(core_axis_name="c", subcore_axis_name="s")
    k = pl.kernel(
        functools.partial(_scat_body, co),
        out_type=jax.ShapeDtypeStruct((2, NN, 32), _f32),
        mesh=mesh,
        compiler_params=pltpu.CompilerParams(use_tc_tiling_on_sc=False),
        scratch_types=[
            pltpu.VMEM((SCC,), jnp.int32),
            pltpu.VMEM((SCC, 32), _f32),
            pltpu.VMEM_SHARED((NN, 32), _f32),
            pltpu.SemaphoreType.DMA,
        ],
    )
    return k(msg, dst, z)


def _combine_body(last, pk_ref, ss_ref, vs_ref,
                  wu1_ref, bu1_ref, wu2_ref, bu2_ref, g_ref, b_ref,
                  so_ref, vo_ref, x_ref):
    pk = pk_ref[...]
    vs0 = vs_ref[0]
    vs1 = vs_ref[1]
    cnt = jnp.maximum(vs0[:, 27:28], 1.0)
    vsum = jnp.concatenate([vs0[:, 0:24], vs1[:, 0:24]], axis=-1)
    v_new = pk[:, SDIM:SDIM + VFD] + vsum / cnt
    p_new = jnp.concatenate(
        [pk[:, 112:115] + vs0[:, 24:27] / cnt,
         jnp.zeros((pk.shape[0], PP - 3), _f32)], axis=-1)
    s = pk[:, 0:SDIM] + jnp.concatenate([ss_ref[0], ss_ref[1]], axis=-1)
    if not last:
        h = jnp.dot(s, wu1_ref[...], preferred_element_type=_f32) + bu1_ref[...]
        h = h * jax.nn.sigmoid(h)
        s = s + jnp.dot(h, wu2_ref[...], preferred_element_type=_f32) + bu2_ref[...]
    so_ref[...] = s
    vo_ref[...] = v_new
    if last:
        x_ref[...] = p_new
    else:
        sn, vn = _norm_pack(s, v_new, None, g_ref[...], b_ref[...])
        z = jnp.zeros((s.shape[0], PK - SDIM - VFD - PP), _f32)
        x_ref[...] = jnp.concatenate([sn, vn, p_new, z], axis=-1)


def _combine(last, pk, ssum, vsum, wu1, bu1, wu2, bu2, g, b):
    kb = functools.partial(_combine_body, last)
    xw = PP if last else PK
    return pl.pallas_call(
        kb,
        grid=(NN // BN,),
        in_specs=[
            pl.BlockSpec((BN, PK), lambda i: (i, 0)),
            pl.BlockSpec((2, BN, 32), lambda i: (0, i, 0)),
            pl.BlockSpec((2, BN, 32), lambda i: (0, i, 0)),
            pl.BlockSpec((SDIM, HID), lambda i: (0, 0)),
            pl.BlockSpec((1, HID), lambda i: (0, 0)),
            pl.BlockSpec((HID, SDIM), lambda i: (0, 0)),
            pl.BlockSpec((1, SDIM), lambda i: (0, 0)),
            pl.BlockSpec((1, SDIM), lambda i: (0, 0)),
            pl.BlockSpec((1, SDIM), lambda i: (0, 0)),
        ],
        out_specs=[
            pl.BlockSpec((BN, SDIM), lambda i: (i, 0)),
            pl.BlockSpec((BN, VFD), lambda i: (i, 0)),
            pl.BlockSpec((BN, xw), lambda i: (i, 0)),
        ],
        out_shape=[
            jax.ShapeDtypeStruct((NN, SDIM), _f32),
            jax.ShapeDtypeStruct((NN, VFD), _f32),
            jax.ShapeDtypeStruct((NN, xw), _f32),
        ],
    )(pk, ssum, vsum, wu1, bu1.reshape(1, HID), wu2, bu2.reshape(1, SDIM),
      g.reshape(1, SDIM), b.reshape(1, SDIM))


def kernel(s, v, p, edge_index_local, d_local, a_local, r_norm_local, e_local,
           edge_index_global, d_global, a_global, r_norm_global, e_global,
           W1, b1, W2, b2, gamma, beta, Wu1, bu1, Wu2, bu2):
    nl = W1.shape[0]
    src = edge_index_global[0].astype(jnp.int32)
    dst = edge_index_global[1].astype(jnp.int32)

    v = v.reshape(NN, VFD)
    p8 = jnp.pad(p, ((0, 0), (0, PP - 3)))

    z1 = jnp.zeros((nl, PK - SDIM - VFD - PP, HID), _f32)
    zv1 = jnp.zeros((nl, VFD + PP, HID), _f32)
    w1s = jnp.concatenate([W1[:, 0:SDIM, :], zv1, z1], axis=1)
    w1d = jnp.concatenate([W1[:, SDIM:2 * SDIM, :], zv1, z1], axis=1)
    w1c = W1[:, 2 * SDIM:2 * SDIM + EDIM, :]
    w1x8 = jnp.concatenate(
        [W1[:, 2 * SDIM + EDIM:, :], jnp.zeros((nl, PP - 2, HID), _f32)],
        axis=1)
    t16 = jnp.tile(jnp.eye(16, dtype=_f32), (1, 3))
    k848 = jnp.concatenate(
        [jnp.kron(jnp.eye(3, dtype=_f32), jnp.ones((1, 16), _f32)),
         jnp.zeros((PP - 3, VFD), _f32)], axis=0)
    sh8 = jnp.pad(jnp.eye(3, dtype=_f32), ((2, PP - 5), (0, PP - 3)))
    d_row = W1[:, 2 * SDIM + EDIM, :]
    a_row = W1[:, 2 * SDIM + EDIM + 1, :]
    w1a8 = jnp.tile(a_row[:, None, :], (1, PP, 1))
    w1d8 = jnp.tile(d_row[:, None, :], (1, PP, 1)) / 8.0
    sd, vd = SDIM, 16
    wp = W2[:, :, sd + 2 * vd:sd + 2 * vd + 1]
    w2r = jnp.concatenate([
        W2[:, :, 0:sd],
        W2[:, :, sd:sd + vd],
        W2[:, :, sd + vd:sd + 2 * vd],
        W2[:, :, sd + 2 * vd + 1:],
        jnp.tile(wp, (1, 1, PP)),
        jnp.zeros((nl, HID, 8), _f32),
    ], axis=-1)
    bp = b2[:, sd + 2 * vd:sd + 2 * vd + 1]
    b2r = jnp.concatenate([
        b2[:, 0:sd], b2[:, sd:sd + vd], b2[:, sd + vd:sd + 2 * vd],
        b2[:, sd + 2 * vd + 1:], jnp.tile(bp, (1, PP)),
        jnp.zeros((nl, 8), _f32),
    ], axis=-1)

    zv = jnp.zeros((NN, 32), _f32)

    e = e_global
    da1 = jnp.concatenate(
        [d_global[:, None], a_global[:, None], r_norm_global,
         jnp.zeros((EE, 3), _f32)], axis=-1)

    pk = _pack0(s, v, p8, gamma[0], beta[0])
    for i in range(nl):
        srow, drow = _gather_edges(pk, src, dst)
        msg, e = _edge_mlp(
            i == 0, srow, drow, e, da1,
            w1s[i], w1d[i], w1c[i], w1x8[i], w1a8[i], w1d8[i],
            b1[i].reshape(1, HID),
            w2r[i], b2r[i].reshape(1, 128), t16, k848, sh8)
        ssum = _scatter32(msg, dst, zv, 0)
        vsum = _scatter32(msg, dst, zv, SDIM)
        last = i == nl - 1
        gi = min(i + 1, nl - 1)
        s, v, pk = _combine(
            last, pk, ssum, vsum,
            Wu1[i], bu1[i], Wu2[i], bu2[i], gamma[gi], beta[gi])

    return (s, v.reshape(NN, 3, 16), e, pk[:, 0:3])
